# baseline (speedup 1.0000x reference)
import jax
import jax.numpy as jnp
from jax import lax
from jax.experimental import pallas as pl

N = 2048


def _masked_sum(v, mask_b):
    vh = v.astype(jnp.bfloat16)
    r1 = v - vh.astype(jnp.float32)
    vm = r1.astype(jnp.bfloat16)
    vl = (r1 - vm.astype(jnp.float32)).astype(jnp.bfloat16)
    v9 = jnp.concatenate([vh, vm, vl], axis=0)
    w9 = jnp.dot(v9, mask_b, preferred_element_type=jnp.float32)
    return w9[0:3, :] + w9[3:6, :] + w9[6:9, :]


def _layer(h0, h1, p):
    hh0 = h0 * p[0, 0] + h1 * p[0, 1]
    hh1 = h0 * p[0, 2] + h1 * p[0, 3]
    s = hh0 * p[0, 4] + hh1 * p[0, 5]
    d = hh0 * p[0, 6] + hh1 * p[0, 7]
    m1 = jnp.max(s)
    e1 = jnp.exp(s - m1)
    e02 = jnp.exp(0.2 * (s - m1))
    q0 = e1 * hh0
    q1 = e1 * hh1
    ones = jnp.ones_like(s)
    A = jnp.concatenate([s, ones], axis=0)
    B = jnp.concatenate([ones, d], axis=0)
    Z = lax.dot_general(A, B, (((0,), (0,)), ((), ())),
                        preferred_element_type=jnp.float32)
    Mt = jnp.where(Z <= 0.0, 1.0, 0.0).astype(jnp.bfloat16)
    Mt2 = jnp.where(Z > 0.0, 1.0, 0.0).astype(jnp.bfloat16)
    Vn = jnp.concatenate([e02, e02 * hh0, e02 * hh1], axis=0)
    Vp = jnp.concatenate([e1, q0, q1], axis=0)
    Wn = _masked_sum(Vn, Mt)
    Wp = _masked_sum(Vp, Mt2)
    b2 = d + m1
    b1 = 0.2 * b2
    L = jnp.maximum(b1, b2)
    f1 = jnp.exp(b1 - L)
    f2 = jnp.exp(b2 - L)
    den = f1 * Wn[0:1, :] + f2 * Wp[0:1, :]
    o0 = (f1 * Wn[1:2, :] + f2 * Wp[1:2, :]) / den + p[0, 8]
    o1 = (f1 * Wn[2:3, :] + f2 * Wp[2:3, :]) / den + p[0, 9]
    return o0, o1


def _gnn_kernel(x_ref, p0_ref, p1_ref, out_ref):
    x0 = x_ref[...]
    xpos = (lax.broadcasted_iota(jnp.int32, (1, N), 1)
            .astype(jnp.float32) - N / 2)
    o0, o1 = _layer(x0, xpos, p0_ref[...])
    o0, o1 = _layer(o0, o1, p1_ref[...])
    out_ref[0:1, :] = o0
    out_ref[1:2, :] = o1


def _pack_params(lin_w, att_src, att_dst, bias):
    v = jnp.concatenate([
        lin_w[0].reshape(2), lin_w[1].reshape(2),
        att_src.reshape(2), att_dst.reshape(2), bias.reshape(2),
        jnp.zeros((6,), jnp.float32)])
    return v.reshape(1, 16)


@jax.jit
def kernel(x, lin_weight_0, src_weight_0, dst_weight_0, bias_weight_0,
           lin_weight_1, src_weight_1, dst_weight_1, bias_weight_1):
    p0 = _pack_params(lin_weight_0, src_weight_0, dst_weight_0, bias_weight_0)
    p1 = _pack_params(lin_weight_1, src_weight_1, dst_weight_1, bias_weight_1)
    out = pl.pallas_call(
        _gnn_kernel,
        out_shape=jax.ShapeDtypeStruct((2, N), jnp.float32),
    )(x.reshape(1, N), p0, p1)
    return out.T
